# baseline (speedup 1.0000x reference)
import functools

import jax
import jax.numpy as jnp
from jax import lax
from jax.experimental import pallas as pl
from jax.experimental.pallas import tpu as pltpu
from jax.experimental.pallas import tpu_sc as plsc

H = 768
FIXED = 1024
EPS = 1e-12
NLANE = 16
NCHUNK = H // NLANE


def _make_sc_kernel(total_rows: int, rows_per_chunk: int):
    info = plsc.get_sparse_core_info()
    num_workers = info.num_cores * info.num_subcores
    rpw = total_rows // num_workers
    n_chunks = rpw // rows_per_chunk
    C = rows_per_chunk
    mesh = plsc.VectorSubcoreMesh(core_axis_name="c", subcore_axis_name="s")

    @functools.partial(
        pl.kernel,
        mesh=mesh,
        out_type=jax.ShapeDtypeStruct((total_rows, H), jnp.float32),
        scratch_types=[
            pltpu.VMEM((C,), jnp.int32),
            pltpu.VMEM((C,), jnp.int32),
            pltpu.VMEM((C,), jnp.int32),
            pltpu.VMEM((C, H), jnp.float32),
            pltpu.VMEM((C, H), jnp.float32),
            pltpu.VMEM((C, H), jnp.float32),
            pltpu.VMEM((H,), jnp.float32),
            pltpu.VMEM((H,), jnp.float32),
            pltpu.SemaphoreType.DMA,
            pltpu.SemaphoreType.DMA,
            pltpu.SemaphoreType.DMA,
        ],
    )
    def sc_kernel(ids_hbm, pidx_hbm, lidx_hbm, word_hbm, pos_hbm, lang_hbm,
                  g_hbm, b_hbm, out_hbm,
                  widx_v, pidx_v, lidx_v, wbuf, pbuf, lbuf, gbuf, bbuf,
                  sem0, sem1, sem2):
        wid = lax.axis_index("s") * info.num_cores + lax.axis_index("c")
        pltpu.sync_copy(g_hbm, gbuf)
        pltpu.sync_copy(b_hbm, bbuf)

        def chunk_body(c, carry):
            base = wid * rpw + c * C
            pltpu.sync_copy(ids_hbm.at[pl.ds(base, C)], widx_v)
            pltpu.sync_copy(pidx_hbm.at[pl.ds(base, C)], pidx_v)
            pltpu.sync_copy(lidx_hbm.at[pl.ds(base, C)], lidx_v)
            cw = pltpu.async_copy(word_hbm.at[widx_v], wbuf, sem0)
            cp = pltpu.async_copy(pos_hbm.at[pidx_v], pbuf, sem1)
            cl = pltpu.async_copy(lang_hbm.at[lidx_v], lbuf, sem2)
            cw.wait()
            cp.wait()
            cl.wait()

            lanes = lax.iota(jnp.int32, NLANE)
            perms = [lanes ^ (1 << k) for k in range(4)]

            def row_body(r, rcarry):
                acc = jnp.zeros((NLANE,), jnp.float32)
                acc2 = jnp.zeros((NLANE,), jnp.float32)
                for j in range(NCHUNK):
                    sl = pl.ds(j * NLANE, NLANE)
                    e = wbuf[r, sl] + pbuf[r, sl] + lbuf[r, sl]
                    wbuf[r, sl] = e
                    acc = acc + e
                    acc2 = acc2 + e * e
                for p in perms:
                    acc = acc + acc.at[p].get(mode="promise_in_bounds")
                    acc2 = acc2 + acc2.at[p].get(mode="promise_in_bounds")
                muv = acc * (1.0 / H)
                v = acc2 * (1.0 / H) - muv * muv + EPS
                yi = jnp.int32(0x5F3759DF) - (
                    lax.bitcast_convert_type(v, jnp.int32) >> 1)
                y = lax.bitcast_convert_type(yi, jnp.float32)
                half_v = v * 0.5
                for _ in range(3):
                    y = y * (1.5 - half_v * y * y)
                for j in range(NCHUNK):
                    sl = pl.ds(j * NLANE, NLANE)
                    e = wbuf[r, sl]
                    wbuf[r, sl] = (e - muv) * y * gbuf[sl] + bbuf[sl]
                return rcarry

            lax.fori_loop(0, C, row_body, 0, unroll=False)
            pltpu.sync_copy(wbuf, out_hbm.at[pl.ds(base, C)])
            return carry

        lax.fori_loop(0, n_chunks, chunk_body, 0, unroll=False)

    return sc_kernel


def kernel(input_ids, position_offset, word_table, pos_table, lang_table,
           ln_gamma, ln_beta):
    if input_ids.ndim == 1:
        input_ids = input_ids[None, :]
    B, S = input_ids.shape

    idx = jnp.arange(S, dtype=jnp.int32)
    shifted = idx + jnp.asarray(position_offset, jnp.int32)
    wrapped = shifted >= FIXED
    pos_ids = jnp.where(wrapped, shifted - FIXED, shifted)
    lang_ids = wrapped.astype(jnp.int32)

    ids_flat = input_ids.reshape(-1).astype(jnp.int32)
    pos_flat = jnp.broadcast_to(pos_ids[None, :], (B, S)).reshape(-1)
    lang_flat = jnp.broadcast_to(lang_ids[None, :], (B, S)).reshape(-1)

    total = B * S
    C = 32
    num_workers = 32
    granularity = num_workers * C
    pad = (-total) % granularity
    if pad:
        ids_flat = jnp.concatenate(
            [ids_flat, jnp.zeros((pad,), jnp.int32)])
        pos_flat = jnp.concatenate(
            [pos_flat, jnp.zeros((pad,), jnp.int32)])
        lang_flat = jnp.concatenate(
            [lang_flat, jnp.zeros((pad,), jnp.int32)])

    sc = _make_sc_kernel(total + pad, C)
    out = sc(ids_flat, pos_flat, lang_flat,
             word_table.astype(jnp.float32),
             pos_table.astype(jnp.float32),
             lang_table.astype(jnp.float32),
             ln_gamma.astype(jnp.float32), ln_beta.astype(jnp.float32))
    if pad:
        out = out[:total]
    return out.reshape(B, S, H)


# speedup vs baseline: 2.1942x; 2.1942x over previous
import functools

import jax
import jax.numpy as jnp
from jax import lax
from jax.experimental import pallas as pl
from jax.experimental.pallas import tpu as pltpu
from jax.experimental.pallas import tpu_sc as plsc

H = 768
FIXED = 1024
EPS = 1e-12
NLANE = 16
NCHUNK = H // NLANE
CS = 8


def _make_sc_kernel(batch: int, s_pad: int):
    info = plsc.get_sparse_core_info()
    num_workers = info.num_cores * info.num_subcores
    spw = s_pad // num_workers
    n_chunks = spw // CS
    n_pairs = n_chunks // 2
    B = batch
    BCS = B * CS
    mesh = plsc.VectorSubcoreMesh(core_axis_name="c", subcore_axis_name="s")

    @functools.partial(
        pl.kernel,
        mesh=mesh,
        out_type=jax.ShapeDtypeStruct((B * s_pad, H), jnp.float32),
        scratch_types=[
            pltpu.VMEM((B * spw,), jnp.int32),
            pltpu.VMEM((spw,), jnp.int32),
            pltpu.VMEM((spw,), jnp.int32),
            pltpu.VMEM((2, BCS, H), jnp.float32),
            pltpu.VMEM((2, CS, H), jnp.float32),
            pltpu.VMEM((2, CS, H), jnp.float32),
            pltpu.VMEM((H,), jnp.float32),
            pltpu.VMEM((H,), jnp.float32),
            pltpu.SemaphoreType.DMA,
            pltpu.SemaphoreType.DMA,
        ],
    )
    def sc_kernel(ids_hbm, pidx_hbm, lidx_hbm, word_hbm, pos_hbm, lang_hbm,
                  g_hbm, b_hbm, out_hbm,
                  widx, pidx, lidx, wbufs, pbufs, lbufs, gbuf, bbuf,
                  sem0, sem1):
        wid = lax.axis_index("s") * info.num_cores + lax.axis_index("c")
        s_base = wid * spw
        pltpu.sync_copy(g_hbm, gbuf)
        pltpu.sync_copy(b_hbm, bbuf)
        for b in range(B):
            pltpu.sync_copy(ids_hbm.at[pl.ds(b * s_pad + s_base, spw)],
                            widx.at[pl.ds(b * spw, spw)])
        pltpu.sync_copy(pidx_hbm.at[pl.ds(s_base, spw)], pidx)
        pltpu.sync_copy(lidx_hbm.at[pl.ds(s_base, spw)], lidx)

        sems = (sem0, sem1)
        bufs = ((wbufs.at[0], pbufs.at[0], lbufs.at[0]),
                (wbufs.at[1], pbufs.at[1], lbufs.at[1]))

        def gather_descs(slot, c):
            wb, pb, lb = bufs[slot]
            sem = sems[slot]
            s0 = c * CS
            ds = []
            for b in range(B):
                ds.append(pltpu.make_async_copy(
                    word_hbm.at[widx.at[pl.ds(b * spw + s0, CS)]],
                    wb.at[pl.ds(b * CS, CS)], sem))
            ds.append(pltpu.make_async_copy(
                pos_hbm.at[pidx.at[pl.ds(s0, CS)]], pb, sem))
            ds.append(pltpu.make_async_copy(
                lang_hbm.at[lidx.at[pl.ds(s0, CS)]], lb, sem))
            return ds

        def start_chunk(slot, c):
            for d in gather_descs(slot, c):
                d.start()

        def wait_chunk(slot, c):
            for d in gather_descs(slot, c):
                d.wait()

        lanes = lax.iota(jnp.int32, NLANE)
        perms = [lanes ^ (1 << k) for k in range(4)]

        def compute_chunk(slot, c):
            wb, pb, lb = bufs[slot]

            def col_body(s_loc, carry):
                acc = [jnp.zeros((NLANE,), jnp.float32) for _ in range(B)]
                acc2 = [jnp.zeros((NLANE,), jnp.float32) for _ in range(B)]
                for j in range(NCHUNK):
                    sl = pl.ds(j * NLANE, NLANE)
                    pla = pb[s_loc, sl] + lb[s_loc, sl]
                    for b in range(B):
                        e = wb[b * CS + s_loc, sl] + pla
                        wb[b * CS + s_loc, sl] = e
                        acc[b] = acc[b] + e
                        acc2[b] = acc2[b] + e * e
                muv = []
                inv = []
                for b in range(B):
                    a1, a2 = acc[b], acc2[b]
                    for p in perms:
                        a1 = a1 + a1.at[p].get(mode="promise_in_bounds")
                        a2 = a2 + a2.at[p].get(mode="promise_in_bounds")
                    m = a1 * (1.0 / H)
                    v = a2 * (1.0 / H) - m * m + EPS
                    yi = jnp.int32(0x5F3759DF) - (
                        lax.bitcast_convert_type(v, jnp.int32) >> 1)
                    y = lax.bitcast_convert_type(yi, jnp.float32)
                    hv = v * 0.5
                    for _ in range(3):
                        y = y * (1.5 - hv * y * y)
                    muv.append(m)
                    inv.append(y)
                for j in range(NCHUNK):
                    sl = pl.ds(j * NLANE, NLANE)
                    g = gbuf[sl]
                    bt = bbuf[sl]
                    for b in range(B):
                        t = inv[b] * g
                        u = bt - muv[b] * t
                        e = wb[b * CS + s_loc, sl]
                        wb[b * CS + s_loc, sl] = e * t + u
                return carry

            lax.fori_loop(0, CS, col_body, 0, unroll=False)

        def write_chunk(slot, c):
            wb, _, _ = bufs[slot]
            s0 = c * CS
            for b in range(B):
                pltpu.sync_copy(
                    wb.at[pl.ds(b * CS, CS)],
                    out_hbm.at[pl.ds(b * s_pad + s_base + s0, CS)])

        start_chunk(0, 0)

        def pair_body(p, carry):
            c0 = 2 * p
            c1 = c0 + 1
            start_chunk(1, c1)
            wait_chunk(0, c0)
            compute_chunk(0, c0)
            write_chunk(0, c0)

            @pl.when(c1 + 1 < n_chunks)
            def _():
                start_chunk(0, c1 + 1)

            wait_chunk(1, c1)
            compute_chunk(1, c1)
            write_chunk(1, c1)
            return carry

        lax.fori_loop(0, n_pairs, pair_body, 0, unroll=False)

    return sc_kernel


def kernel(input_ids, position_offset, word_table, pos_table, lang_table,
           ln_gamma, ln_beta):
    if input_ids.ndim == 1:
        input_ids = input_ids[None, :]
    B, S = input_ids.shape

    num_workers = 32
    granularity = num_workers * CS * 2
    s_pad = -(-S // granularity) * granularity
    pad = s_pad - S

    idx = jnp.arange(s_pad, dtype=jnp.int32)
    shifted = idx + jnp.asarray(position_offset, jnp.int32)
    wrapped = jnp.logical_and(shifted >= FIXED, idx < S)
    pos_ids = jnp.where(wrapped, shifted - FIXED, shifted)
    pos_ids = jnp.where(idx < S, pos_ids, 0)
    lang_ids = wrapped.astype(jnp.int32)

    ids = input_ids.astype(jnp.int32)
    if pad:
        ids = jnp.pad(ids, ((0, 0), (0, pad)))
    ids_flat = ids.reshape(-1)

    sc = _make_sc_kernel(B, s_pad)
    out = sc(ids_flat, pos_ids, lang_ids,
             word_table.astype(jnp.float32),
             pos_table.astype(jnp.float32),
             lang_table.astype(jnp.float32),
             ln_gamma.astype(jnp.float32), ln_beta.astype(jnp.float32))
    out = out.reshape(B, s_pad, H)
    if pad:
        out = out[:, :S]
    return out
